# Initial kernel scaffold; baseline (speedup 1.0000x reference)
import jax
import jax.numpy as jnp
from jax.experimental import pallas as pl


def kernel(x, edge_index, W_msg, W_self, b):
    raise NotImplementedError("write your pallas kernel here")


# speedup vs baseline: 6.9117x; 6.9117x over previous
import functools

import jax
import jax.numpy as jnp
from jax import lax
from jax.experimental import pallas as pl
from jax.experimental.pallas import tpu as pltpu
from jax.experimental.pallas import tpu_sc as plsc

_NC = 2
_NS = 16
_C = 128
_N_PAD = 10240


def _sc_segment_sum(x, src, dst):
  n_nodes, d = x.shape
  n_pad = _N_PAD
  n_edges = src.shape[0]
  nchunks = n_edges // _C
  assert nchunks * _C == n_edges
  nw = _NC * _NS
  per, rem = divmod(nchunks, nw)
  rows_per_tile = n_pad // _NS
  assert rows_per_tile % _C == 0

  mesh = plsc.VectorSubcoreMesh(
      core_axis_name="c", subcore_axis_name="s",
      num_cores=_NC, num_subcores=_NS)

  @functools.partial(
      pl.kernel,
      out_type=jax.ShapeDtypeStruct((_NC, n_pad, d), jnp.float32),
      mesh=mesh,
      scratch_types=[
          pltpu.VMEM_SHARED((n_pad, d), jnp.float32),
          pltpu.VMEM((_C,), jnp.int32),
          pltpu.VMEM((_C,), jnp.int32),
          pltpu.VMEM((_C, d), jnp.float32),
          pltpu.SemaphoreType.DMA,
      ],
  )
  def k(x_hbm, src_hbm, dst_hbm, out_hbm, acc, sidx, didx, rows, sem):
    cid = lax.axis_index("c")
    tid = lax.axis_index("s")

    def zrow(i, _):
      for j in range(d // 16):
        rows[i, pl.ds(j * 16, 16)] = jnp.zeros((16,), jnp.float32)
      return 0
    lax.fori_loop(0, _C, zrow, 0)

    r0 = tid * rows_per_tile
    for kk in range(rows_per_tile // _C):
      pltpu.sync_copy(rows, acc.at[pl.ds(r0 + kk * _C, _C)])
    plsc.subcore_barrier()

    w = cid * _NS + tid
    cbase = w * per + jnp.minimum(w, rem)
    ccount = per + (w < rem).astype(jnp.int32)

    def body(c, _):
      e0 = pl.multiple_of(c * _C, _C)
      pltpu.sync_copy(src_hbm.at[pl.ds(e0, _C)], sidx)
      pltpu.sync_copy(dst_hbm.at[pl.ds(e0, _C)], didx)
      pltpu.async_copy(x_hbm.at[sidx], rows, sem).wait()
      pltpu.sync_copy(rows, acc.at[didx], add=True)
      return 0
    lax.fori_loop(cbase, cbase + ccount, body, 0)
    plsc.subcore_barrier()

    pltpu.sync_copy(acc.at[pl.ds(r0, rows_per_tile)],
                    out_hbm.at[cid, pl.ds(r0, rows_per_tile)])

  return k(x, src, dst)


def _tc_body(p_ref, x_ref, wm_ref, ws_ref, b_ref, o_ref):
  agg = p_ref[0] + p_ref[1]
  h = jnp.dot(agg, wm_ref[...], preferred_element_type=jnp.float32)
  h = h + jnp.dot(x_ref[...], ws_ref[...], preferred_element_type=jnp.float32)
  o_ref[...] = jnp.maximum(h + b_ref[...], 0.0)


def _tc_dense(partials, x, w_msg, w_self, b2):
  n, d = x.shape
  br = 2000
  grid = (n // br,)
  return pl.pallas_call(
      _tc_body,
      grid=grid,
      in_specs=[
          pl.BlockSpec((_NC, br, d), lambda i: (0, i, 0)),
          pl.BlockSpec((br, d), lambda i: (i, 0)),
          pl.BlockSpec((d, d), lambda i: (0, 0)),
          pl.BlockSpec((d, d), lambda i: (0, 0)),
          pl.BlockSpec((1, d), lambda i: (0, 0)),
      ],
      out_specs=pl.BlockSpec((br, d), lambda i: (i, 0)),
      out_shape=jax.ShapeDtypeStruct((n, d), jnp.float32),
  )(partials, x, w_msg, w_self, b2)


@jax.jit
def kernel(x, edge_index, W_msg, W_self, b):
  ei = edge_index.astype(jnp.int32)
  src = ei[0]
  dst = ei[1]
  partials = _sc_segment_sum(x, src, dst)
  return _tc_dense(partials, x, W_msg, W_self, b.reshape(1, -1))
